# baseline (speedup 1.0000x reference)
import functools

import jax
import jax.numpy as jnp
from jax.experimental import pallas as pl
from jax.experimental.pallas import tpu as pltpu

_BN = 1024
_BK = 1024


def _dist_argmin_kernel(x_ref, ct_ref, idx_ref, dist_ref, max_sc, kwin_sc,
                        swin_sc):
    k = pl.program_id(1)
    nk = pl.num_programs(1)

    @pl.when(k == 0)
    def _init():
        max_sc[...] = jnp.full(max_sc.shape, -jnp.inf, jnp.float32)
        kwin_sc[...] = jnp.zeros(kwin_sc.shape, jnp.int32)

    x = x_ref[...]
    ct = ct_ref[...]
    dot = jnp.dot(x, ct, preferred_element_type=jnp.float32)
    half_c2 = 0.5 * jnp.sum(ct * ct, axis=0, keepdims=True)
    score = dot - half_c2

    bmax = jnp.max(score, axis=1, keepdims=True)
    improved = bmax > max_sc[...]
    max_sc[...] = jnp.where(improved, bmax, max_sc[...])
    kwin_sc[...] = jnp.where(improved, k, kwin_sc[...])
    swin_sc[...] = jnp.where(improved, score, swin_sc[...])

    @pl.when(k == nk - 1)
    def _write():
        bk = score.shape[1]
        swin = swin_sc[...]
        iota = jax.lax.broadcasted_iota(jnp.int32, swin.shape, 1)
        local = jnp.min(jnp.where(swin == max_sc[...], iota, bk), axis=1,
                        keepdims=True)
        idx_ref[...] = kwin_sc[...] * bk + local
        x2 = jnp.sum(x * x, axis=1, keepdims=True)
        dist_ref[...] = jnp.sqrt(jnp.maximum(x2 - 2.0 * max_sc[...], 0.0))


@functools.partial(jax.jit, static_argnames=())
def _assign(X, ct):
    n, d = X.shape
    kk = ct.shape[1]
    bn, bk = _BN, _BK
    grid = (n // bn, kk // bk)
    idx2, dist2 = pl.pallas_call(
        _dist_argmin_kernel,
        grid=grid,
        in_specs=[
            pl.BlockSpec((bn, d), lambda i, k: (i, 0)),
            pl.BlockSpec((d, bk), lambda i, k: (0, k)),
        ],
        out_specs=[
            pl.BlockSpec((bn, 1), lambda i, k: (i, 0)),
            pl.BlockSpec((bn, 1), lambda i, k: (i, 0)),
        ],
        out_shape=[
            jax.ShapeDtypeStruct((n, 1), jnp.int32),
            jax.ShapeDtypeStruct((n, 1), jnp.float32),
        ],
        scratch_shapes=[
            pltpu.VMEM((bn, 1), jnp.float32),
            pltpu.VMEM((bn, 1), jnp.int32),
            pltpu.VMEM((bn, bk), jnp.float32),
        ],
        compiler_params=pltpu.CompilerParams(
            dimension_semantics=("parallel", "arbitrary"),
        ),
    )(X, ct)
    return idx2[:, 0], dist2[:, 0]


def kernel(X, codebook, return_dist):
    idx, dist = _assign(X, codebook.T)
    dist = dist * jnp.asarray(return_dist, dist.dtype)
    return (idx, dist)


# speedup vs baseline: 1.6016x; 1.6016x over previous
import functools

import jax
import jax.numpy as jnp
from jax.experimental import pallas as pl
from jax.experimental.pallas import tpu as pltpu

_BN = 1024
_BK = 1024
_LANES = 128


def _dist_argmin_kernel(x_ref, ct_ref, idx_ref, dist_ref, rmax_sc, rarg_sc):
    k = pl.program_id(1)
    nk = pl.num_programs(1)

    @pl.when(k == 0)
    def _init():
        rmax_sc[...] = jnp.full(rmax_sc.shape, -jnp.inf, jnp.float32)
        rarg_sc[...] = jnp.zeros(rarg_sc.shape, jnp.int32)

    x = x_ref[...]
    ct = ct_ref[...]
    dot = jnp.dot(x, ct, preferred_element_type=jnp.float32)
    half_c2 = 0.5 * jnp.sum(ct * ct, axis=0, keepdims=True)
    score = dot - half_c2

    bk = score.shape[1]
    lanes = _LANES
    lane_iota = jax.lax.broadcasted_iota(jnp.int32, (1, lanes), 1)
    rmax = rmax_sc[...]
    rarg = rarg_sc[...]
    for g in range(bk // lanes):
        sg = score[:, g * lanes:(g + 1) * lanes]
        cand = lane_iota + (k * bk + g * lanes)
        gt = sg > rmax
        rarg = jnp.where(gt, cand, rarg)
        rmax = jnp.maximum(rmax, sg)
    rmax_sc[...] = rmax
    rarg_sc[...] = rarg

    @pl.when(k == nk - 1)
    def _write():
        m = jnp.max(rmax, axis=1, keepdims=True)
        cand = jnp.where(rmax == m, rarg, jnp.iinfo(jnp.int32).max)
        idx_ref[...] = jnp.min(cand, axis=1, keepdims=True)
        x2 = jnp.sum(x * x, axis=1, keepdims=True)
        dist_ref[...] = jnp.sqrt(jnp.maximum(x2 - 2.0 * m, 0.0))


@functools.partial(jax.jit, static_argnames=())
def _assign(X, ct):
    n, d = X.shape
    kk = ct.shape[1]
    bn, bk = _BN, _BK
    grid = (n // bn, kk // bk)
    idx2, dist2 = pl.pallas_call(
        _dist_argmin_kernel,
        grid=grid,
        in_specs=[
            pl.BlockSpec((bn, d), lambda i, k: (i, 0)),
            pl.BlockSpec((d, bk), lambda i, k: (0, k)),
        ],
        out_specs=[
            pl.BlockSpec((bn, 1), lambda i, k: (i, 0)),
            pl.BlockSpec((bn, 1), lambda i, k: (i, 0)),
        ],
        out_shape=[
            jax.ShapeDtypeStruct((n, 1), jnp.int32),
            jax.ShapeDtypeStruct((n, 1), jnp.float32),
        ],
        scratch_shapes=[
            pltpu.VMEM((bn, _LANES), jnp.float32),
            pltpu.VMEM((bn, _LANES), jnp.int32),
        ],
        compiler_params=pltpu.CompilerParams(
            dimension_semantics=("parallel", "arbitrary"),
        ),
    )(X, ct)
    return idx2[:, 0], dist2[:, 0]


def kernel(X, codebook, return_dist):
    idx, dist = _assign(X, codebook.T)
    dist = dist * jnp.asarray(return_dist, dist.dtype)
    return (idx, dist)


# speedup vs baseline: 2.0312x; 1.2682x over previous
import functools

import jax
import jax.numpy as jnp
from jax.experimental import pallas as pl
from jax.experimental.pallas import tpu as pltpu

_BN = 2048
_BK = 2048
_LANES = 128


def _dist_argmin_kernel(x_ref, ct_ref, idx_ref, dist_ref, rmax_sc, rarg_sc):
    k = pl.program_id(1)
    nk = pl.num_programs(1)

    @pl.when(k == 0)
    def _init():
        rmax_sc[...] = jnp.full(rmax_sc.shape, -jnp.inf, jnp.float32)
        rarg_sc[...] = jnp.zeros(rarg_sc.shape, jnp.int32)

    x = x_ref[...]
    ct = ct_ref[...]
    dot = jnp.dot(x, ct, preferred_element_type=jnp.float32)
    half_c2 = 0.5 * jnp.sum(ct * ct, axis=0, keepdims=True)
    score = dot - half_c2

    bk = score.shape[1]
    lanes = _LANES
    lane_iota = jax.lax.broadcasted_iota(jnp.int32, (1, lanes), 1)
    rmax = rmax_sc[...]
    rarg = rarg_sc[...]
    for g in range(bk // lanes):
        sg = score[:, g * lanes:(g + 1) * lanes]
        cand = lane_iota + (k * bk + g * lanes)
        gt = sg > rmax
        rarg = jnp.where(gt, cand, rarg)
        rmax = jnp.maximum(rmax, sg)
    rmax_sc[...] = rmax
    rarg_sc[...] = rarg

    @pl.when(k == nk - 1)
    def _write():
        m = jnp.max(rmax, axis=1, keepdims=True)
        cand = jnp.where(rmax == m, rarg, jnp.iinfo(jnp.int32).max)
        idx_ref[...] = jnp.min(cand, axis=1, keepdims=True)
        x2 = jnp.sum(x * x, axis=1, keepdims=True)
        dist_ref[...] = jnp.sqrt(jnp.maximum(x2 - 2.0 * m, 0.0))


@functools.partial(jax.jit, static_argnames=())
def _assign(X, ct):
    n, d = X.shape
    kk = ct.shape[1]
    bn, bk = _BN, _BK
    grid = (n // bn, kk // bk)
    idx2, dist2 = pl.pallas_call(
        _dist_argmin_kernel,
        grid=grid,
        in_specs=[
            pl.BlockSpec((bn, d), lambda i, k: (i, 0)),
            pl.BlockSpec((d, bk), lambda i, k: (0, k)),
        ],
        out_specs=[
            pl.BlockSpec((bn, 1), lambda i, k: (i, 0)),
            pl.BlockSpec((bn, 1), lambda i, k: (i, 0)),
        ],
        out_shape=[
            jax.ShapeDtypeStruct((n, 1), jnp.int32),
            jax.ShapeDtypeStruct((n, 1), jnp.float32),
        ],
        scratch_shapes=[
            pltpu.VMEM((bn, _LANES), jnp.float32),
            pltpu.VMEM((bn, _LANES), jnp.int32),
        ],
        compiler_params=pltpu.CompilerParams(
            dimension_semantics=("parallel", "arbitrary"),
        ),
    )(X, ct)
    return idx2[:, 0], dist2[:, 0]


def kernel(X, codebook, return_dist):
    idx, dist = _assign(X, codebook.T)
    dist = dist * jnp.asarray(return_dist, dist.dtype)
    return (idx, dist)


# speedup vs baseline: 2.1009x; 1.0343x over previous
import functools

import jax
import jax.numpy as jnp
from jax.experimental import pallas as pl
from jax.experimental.pallas import tpu as pltpu

_BN = 4096
_BK = 2048
_LANES = 128


def _dist_argmin_kernel(x_ref, ct_ref, idx_ref, dist_ref, rmax_sc, rarg_sc):
    k = pl.program_id(1)
    nk = pl.num_programs(1)

    @pl.when(k == 0)
    def _init():
        rmax_sc[...] = jnp.full(rmax_sc.shape, -jnp.inf, jnp.float32)
        rarg_sc[...] = jnp.zeros(rarg_sc.shape, jnp.int32)

    x = x_ref[...]
    ct = ct_ref[...]
    dot = jnp.dot(x, ct, preferred_element_type=jnp.float32)
    half_c2 = 0.5 * jnp.sum(ct * ct, axis=0, keepdims=True)
    score = dot - half_c2

    bk = score.shape[1]
    lanes = _LANES
    lane_iota = jax.lax.broadcasted_iota(jnp.int32, (1, lanes), 1)
    rmax = rmax_sc[...]
    rarg = rarg_sc[...]
    for g in range(bk // lanes):
        sg = score[:, g * lanes:(g + 1) * lanes]
        cand = lane_iota + (k * bk + g * lanes)
        gt = sg > rmax
        rarg = jnp.where(gt, cand, rarg)
        rmax = jnp.maximum(rmax, sg)
    rmax_sc[...] = rmax
    rarg_sc[...] = rarg

    @pl.when(k == nk - 1)
    def _write():
        m = jnp.max(rmax, axis=1, keepdims=True)
        cand = jnp.where(rmax == m, rarg, jnp.iinfo(jnp.int32).max)
        idx_ref[...] = jnp.min(cand, axis=1, keepdims=True)
        x2 = jnp.sum(x * x, axis=1, keepdims=True)
        dist_ref[...] = jnp.sqrt(jnp.maximum(x2 - 2.0 * m, 0.0))


@functools.partial(jax.jit, static_argnames=())
def _assign(X, ct):
    n, d = X.shape
    kk = ct.shape[1]
    bn, bk = _BN, _BK
    grid = (n // bn, kk // bk)
    idx2, dist2 = pl.pallas_call(
        _dist_argmin_kernel,
        grid=grid,
        in_specs=[
            pl.BlockSpec((bn, d), lambda i, k: (i, 0)),
            pl.BlockSpec((d, bk), lambda i, k: (0, k)),
        ],
        out_specs=[
            pl.BlockSpec((bn, 1), lambda i, k: (i, 0)),
            pl.BlockSpec((bn, 1), lambda i, k: (i, 0)),
        ],
        out_shape=[
            jax.ShapeDtypeStruct((n, 1), jnp.int32),
            jax.ShapeDtypeStruct((n, 1), jnp.float32),
        ],
        scratch_shapes=[
            pltpu.VMEM((bn, _LANES), jnp.float32),
            pltpu.VMEM((bn, _LANES), jnp.int32),
        ],
        compiler_params=pltpu.CompilerParams(
            dimension_semantics=("parallel", "arbitrary"),
        ),
    )(X, ct)
    return idx2[:, 0], dist2[:, 0]


def kernel(X, codebook, return_dist):
    idx, dist = _assign(X, codebook.T)
    dist = dist * jnp.asarray(return_dist, dist.dtype)
    return (idx, dist)
